# baseline (speedup 1.0000x reference)
import functools

import jax
import jax.numpy as jnp
from jax import lax
from jax.experimental import pallas as pl
from jax.experimental.pallas import tpu as pltpu
from jax.experimental.pallas import tpu_sc as plsc

N = 50000
D = 128
DW = D // 2
K = 10

NC = 2
NS = 16
NW = NC * NS

BN = 12
S0 = 210
S1 = 54
C_PER_W0 = BN * S0
C_PER_W1 = BN * S1
NPAD = NS * (C_PER_W0 + C_PER_W1)
IDX_PAD = (S0 - S1) * BN * K

_mesh = plsc.VectorSubcoreMesh(core_axis_name="c", subcore_axis_name="s")


@functools.partial(
    pl.kernel,
    out_type=jax.ShapeDtypeStruct((NPAD * D,), jnp.float32),
    mesh=_mesh,
    compiler_params=pltpu.CompilerParams(use_tc_tiling_on_sc=False),
    scratch_types=[
        pltpu.VMEM((S0 * BN * K,), jnp.int32),
        pltpu.VMEM((3, BN * K, DW), jnp.int32),
        pltpu.VMEM((3, BN * D), jnp.float32),
        pltpu.SemaphoreType.DMA,
        pltpu.SemaphoreType.DMA,
        pltpu.SemaphoreType.DMA,
        pltpu.SemaphoreType.DMA,
        pltpu.SemaphoreType.DMA,
        pltpu.SemaphoreType.DMA,
    ],
)
def _neighbor_sum(h_hbm, idx_hbm, out_hbm, idx_v, rows_v, out_v,
                  si0, si1, si2, so0, so1, so2):
    c = lax.axis_index("c")
    s = lax.axis_index("s")
    base_n = jnp.where(c == 0, s * C_PER_W0,
                       NS * C_PER_W0 + s * C_PER_W1)
    my_steps = jnp.where(c == 0, S0, S1)
    base = base_n * D
    sin = (si0, si1, si2)
    sout = (so0, so1, so2)

    himask = jnp.int32(-65536)

    pltpu.sync_copy(idx_hbm.at[pl.ds(base_n * K, S0 * BN * K)], idx_v)
    for p in range(2):
        pltpu.async_copy(h_hbm.at[idx_v.at[pl.ds(p * (BN * K), BN * K)]],
                         rows_v.at[p], sin[p])

    @pl.loop(0, my_steps, step=3)
    def _steps(g):
        for b in range(3):
            gb = g + b
            nb = (b + 2) % 3

            @pl.when(gb + 2 < my_steps)
            def _():
                pltpu.async_copy(
                    h_hbm.at[idx_v.at[pl.ds((gb + 2) * (BN * K), BN * K)]],
                    rows_v.at[nb], sin[nb])

            pltpu.make_async_copy(
                h_hbm.at[idx_v.at[pl.ds(gb * (BN * K), BN * K)]],
                rows_v.at[b], sin[b]).wait()

            @pl.when(gb >= 3)
            def _():
                pltpu.make_async_copy(
                    out_v.at[b],
                    out_hbm.at[pl.ds(base + (gb - 3) * (BN * D), BN * D)],
                    sout[b]).wait()

            for nn in range(BN):
                r0 = nn * K
                for gr in range(DW // 16):
                    sl = pl.ds(gr * 16, 16)
                    w = rows_v[b, r0, sl]
                    acc_l = lax.bitcast_convert_type(w << 16, jnp.float32)
                    acc_h = lax.bitcast_convert_type(w & himask, jnp.float32)
                    for k in range(1, K):
                        w = rows_v[b, r0 + k, sl]
                        acc_l = acc_l + lax.bitcast_convert_type(
                            w << 16, jnp.float32)
                        acc_h = acc_h + lax.bitcast_convert_type(
                            w & himask, jnp.float32)
                    out_v[b, pl.ds(nn * D + gr * 16, 16)] = acc_l
                    out_v[b, pl.ds(nn * D + DW + gr * 16, 16)] = acc_h

            pltpu.async_copy(out_v.at[b],
                             out_hbm.at[pl.ds(base + gb * (BN * D), BN * D)],
                             sout[b])

    for p in range(3):
        gb = my_steps - 3 + p
        b = p
        pltpu.make_async_copy(out_v.at[b],
                              out_hbm.at[pl.ds(base + gb * (BN * D), BN * D)],
                              sout[b]).wait()


_RB = 2000
_HB = _RB // 2
_GRID = N // _RB


def _pack_rows(y):
    u = lax.bitcast_convert_type(y, jnp.uint32)
    r = (u + jnp.uint32(0x7FFF) + ((u >> 16) & jnp.uint32(1))) >> 16
    w = r[:, :DW] | (r[:, DW:] << 16)
    wp = jnp.concatenate([w[:_HB], w[_HB:]], axis=1)
    return lax.bitcast_convert_type(wp, jnp.int32)


def _tc_body(apply_ln, want_pack, h_ref, a_ref, ws_ref, wn_ref, b_ref,
             g_ref, bb_ref, *out_refs):
    y = jnp.dot(h_ref[...], ws_ref[...], preferred_element_type=jnp.float32)
    y = y + jnp.dot(a_ref[...], wn_ref[...],
                    preferred_element_type=jnp.float32)
    y = y + b_ref[...]
    if apply_ln:
        mu = jnp.mean(y, axis=-1, keepdims=True)
        var = jnp.mean((y - mu) * (y - mu), axis=-1, keepdims=True)
        ln = (y - mu) * lax.rsqrt(var + 1e-5) * g_ref[...] + bb_ref[...]
        y = y + jnp.maximum(ln, 0.0)
    out_refs[0][...] = y
    if want_pack:
        out_refs[1][...] = _pack_rows(y)


def _tc_layer(h, aggsum, ws, wn, bias, g, bb, apply_ln, want_pack):
    row_spec = pl.BlockSpec((_RB, D), lambda i: (i, 0))
    full_spec = pl.BlockSpec((D, D), lambda i: (0, 0))
    vec_spec = pl.BlockSpec((1, D), lambda i: (0, 0))
    out_shape = [jax.ShapeDtypeStruct((N, D), jnp.float32)]
    out_specs = [row_spec]
    if want_pack:
        out_shape.append(jax.ShapeDtypeStruct((N // 2, D), jnp.int32))
        out_specs.append(pl.BlockSpec((_HB, D), lambda i: (i, 0)))
    res = pl.pallas_call(
        functools.partial(_tc_body, apply_ln, want_pack),
        grid=(_GRID,),
        in_specs=[row_spec, row_spec, full_spec, full_spec, vec_spec,
                  vec_spec, vec_spec],
        out_specs=out_specs,
        out_shape=out_shape,
    )(h, aggsum, ws, wn, bias, g, bb)
    return res if want_pack else (res[0], None)


def _pack_x(x):
    u = lax.bitcast_convert_type(x, jnp.uint32)
    r = (u + jnp.uint32(0x7FFF) + ((u >> 16) & jnp.uint32(1))) >> 16
    w = r[:, :DW] | (r[:, DW:] << 16)
    ws_ = w.reshape(_GRID, _RB, DW)
    wp = jnp.concatenate([ws_[:, :_HB], ws_[:, _HB:]], axis=2)
    return lax.bitcast_convert_type(wp.reshape(N // 2, D), jnp.int32)


def kernel(x, neighbor_indices, Wl0, bl0, Wl1, bl1, Wl2, bl2,
           ln0_g, ln0_b, ln1_g, ln1_b):
    v = neighbor_indices
    blk = v // _RB
    rem = v % _RB
    tv = (blk * _HB + rem % _HB) * 2 + rem // _HB
    idx = jnp.pad(tv, ((0, NPAD - N), (0, 0))).reshape(-1)
    idx = jnp.pad(idx, (0, IDX_PAD))

    Wl = [Wl0, Wl1, Wl2]
    bl = [bl0, bl1, bl2]
    ln_g = [ln0_g, ln1_g]
    ln_b = [ln0_b, ln1_b]

    one = jnp.ones((1, D), jnp.float32)
    zero = jnp.zeros((1, D), jnp.float32)

    h = x
    hp = _pack_x(x)
    for i in range(3):
        ws = Wl[i][:, :D].T
        wn = Wl[i][:, D:].T * (1.0 / K)
        bias = bl[i].reshape(1, D)
        table = hp.reshape(N, DW)
        aggsum = _neighbor_sum(table, idx).reshape(NPAD, D)
        if i < 2:
            h, hp = _tc_layer(h, aggsum, ws, wn, bias, ln_g[i].reshape(1, D),
                              ln_b[i].reshape(1, D), True, True)
        else:
            h, _ = _tc_layer(h, aggsum, ws, wn, bias, one, zero, False,
                             False)
    return h


# speedup vs baseline: 1.0302x; 1.0302x over previous
import functools

import jax
import jax.numpy as jnp
from jax import lax
from jax.experimental import pallas as pl
from jax.experimental.pallas import tpu as pltpu
from jax.experimental.pallas import tpu_sc as plsc

N = 50000
D = 128
DW = D // 2
K = 10

NC = 2
NS = 16
NW = NC * NS

BN = 12
S0 = 214
S1 = 50
C_PER_W0 = BN * S0
C_PER_W1 = BN * S1
NPAD = NS * (C_PER_W0 + C_PER_W1)
IDX_PAD = (S0 - S1) * BN * K

_mesh = plsc.VectorSubcoreMesh(core_axis_name="c", subcore_axis_name="s")


@functools.partial(
    pl.kernel,
    out_type=jax.ShapeDtypeStruct((NPAD * D,), jnp.float32),
    mesh=_mesh,
    compiler_params=pltpu.CompilerParams(use_tc_tiling_on_sc=False),
    scratch_types=[
        pltpu.VMEM((S0 * BN * K,), jnp.int32),
        pltpu.VMEM((2, BN * K, DW), jnp.int32),
        pltpu.VMEM((2, BN * D), jnp.float32),
        pltpu.SemaphoreType.DMA,
        pltpu.SemaphoreType.DMA,
        pltpu.SemaphoreType.DMA,
        pltpu.SemaphoreType.DMA,
    ],
)
def _neighbor_sum(h_hbm, idx_hbm, out_hbm, idx_v, rows_v, out_v,
                  si0, si1, so0, so1):
    c = lax.axis_index("c")
    s = lax.axis_index("s")
    base_n = jnp.where(c == 0, s * C_PER_W0,
                       NS * C_PER_W0 + s * C_PER_W1)
    my_steps = jnp.where(c == 0, S0, S1)
    base = base_n * D
    sin = (si0, si1)
    sout = (so0, so1)

    himask = jnp.int32(-65536)

    pltpu.sync_copy(idx_hbm.at[pl.ds(base_n * K, S0 * BN * K)], idx_v)
    pltpu.async_copy(h_hbm.at[idx_v.at[pl.ds(0, BN * K)]], rows_v.at[0],
                     sin[0])

    @pl.loop(0, my_steps, step=2)
    def _steps(g):
        for b in range(2):
            gb = g + b
            nb = (b + 1) % 2

            @pl.when(gb + 1 < my_steps)
            def _():
                pltpu.async_copy(
                    h_hbm.at[idx_v.at[pl.ds((gb + 1) * (BN * K), BN * K)]],
                    rows_v.at[nb], sin[nb])

            pltpu.make_async_copy(
                h_hbm.at[idx_v.at[pl.ds(gb * (BN * K), BN * K)]],
                rows_v.at[b], sin[b]).wait()

            @pl.when(gb >= 2)
            def _():
                pltpu.make_async_copy(
                    out_v.at[b],
                    out_hbm.at[pl.ds(base + (gb - 2) * (BN * D), BN * D)],
                    sout[b]).wait()

            for nn in range(BN):
                r0 = nn * K
                for gr in range(DW // 16):
                    sl = pl.ds(gr * 16, 16)
                    w = rows_v[b, r0, sl]
                    acc_l = lax.bitcast_convert_type(w << 16, jnp.float32)
                    acc_h = lax.bitcast_convert_type(w & himask, jnp.float32)
                    for k in range(1, K):
                        w = rows_v[b, r0 + k, sl]
                        acc_l = acc_l + lax.bitcast_convert_type(
                            w << 16, jnp.float32)
                        acc_h = acc_h + lax.bitcast_convert_type(
                            w & himask, jnp.float32)
                    out_v[b, pl.ds(nn * D + gr * 16, 16)] = acc_l
                    out_v[b, pl.ds(nn * D + DW + gr * 16, 16)] = acc_h

            pltpu.async_copy(out_v.at[b],
                             out_hbm.at[pl.ds(base + gb * (BN * D), BN * D)],
                             sout[b])

    for b in range(2):
        gb = my_steps - 2 + b
        pltpu.make_async_copy(out_v.at[b],
                              out_hbm.at[pl.ds(base + gb * (BN * D), BN * D)],
                              sout[b]).wait()


_RB = 2000
_HB = _RB // 2
_GRID = N // _RB


def _pack_rows(y):
    u = lax.bitcast_convert_type(y, jnp.uint32)
    r = (u + jnp.uint32(0x7FFF) + ((u >> 16) & jnp.uint32(1))) >> 16
    w = r[:, :DW] | (r[:, DW:] << 16)
    wp = jnp.concatenate([w[:_HB], w[_HB:]], axis=1)
    return lax.bitcast_convert_type(wp, jnp.int32)


def _tc_body(apply_ln, want_pack, h_ref, a_ref, ws_ref, wn_ref, b_ref,
             g_ref, bb_ref, *out_refs):
    y = jnp.dot(h_ref[...], ws_ref[...], preferred_element_type=jnp.float32)
    y = y + jnp.dot(a_ref[...], wn_ref[...],
                    preferred_element_type=jnp.float32)
    y = y + b_ref[...]
    if apply_ln:
        mu = jnp.mean(y, axis=-1, keepdims=True)
        var = jnp.mean((y - mu) * (y - mu), axis=-1, keepdims=True)
        ln = (y - mu) * lax.rsqrt(var + 1e-5) * g_ref[...] + bb_ref[...]
        y = y + jnp.maximum(ln, 0.0)
    out_refs[0][...] = y
    if want_pack:
        out_refs[1][...] = _pack_rows(y)


def _tc_layer(h, aggsum, ws, wn, bias, g, bb, apply_ln, want_pack):
    row_spec = pl.BlockSpec((_RB, D), lambda i: (i, 0))
    full_spec = pl.BlockSpec((D, D), lambda i: (0, 0))
    vec_spec = pl.BlockSpec((1, D), lambda i: (0, 0))
    out_shape = [jax.ShapeDtypeStruct((N, D), jnp.float32)]
    out_specs = [row_spec]
    if want_pack:
        out_shape.append(jax.ShapeDtypeStruct((N // 2, D), jnp.int32))
        out_specs.append(pl.BlockSpec((_HB, D), lambda i: (i, 0)))
    res = pl.pallas_call(
        functools.partial(_tc_body, apply_ln, want_pack),
        grid=(_GRID,),
        in_specs=[row_spec, row_spec, full_spec, full_spec, vec_spec,
                  vec_spec, vec_spec],
        out_specs=out_specs,
        out_shape=out_shape,
    )(h, aggsum, ws, wn, bias, g, bb)
    return res if want_pack else (res[0], None)


def _pack_x(x):
    u = lax.bitcast_convert_type(x, jnp.uint32)
    r = (u + jnp.uint32(0x7FFF) + ((u >> 16) & jnp.uint32(1))) >> 16
    w = r[:, :DW] | (r[:, DW:] << 16)
    ws_ = w.reshape(_GRID, _RB, DW)
    wp = jnp.concatenate([ws_[:, :_HB], ws_[:, _HB:]], axis=2)
    return lax.bitcast_convert_type(wp.reshape(N // 2, D), jnp.int32)


def kernel(x, neighbor_indices, Wl0, bl0, Wl1, bl1, Wl2, bl2,
           ln0_g, ln0_b, ln1_g, ln1_b):
    v = neighbor_indices
    blk = v // _RB
    rem = v % _RB
    tv = (blk * _HB + rem % _HB) * 2 + rem // _HB
    idx = jnp.pad(tv, ((0, NPAD - N), (0, 0))).reshape(-1)
    idx = jnp.pad(idx, (0, IDX_PAD))

    Wl = [Wl0, Wl1, Wl2]
    bl = [bl0, bl1, bl2]
    ln_g = [ln0_g, ln1_g]
    ln_b = [ln0_b, ln1_b]

    one = jnp.ones((1, D), jnp.float32)
    zero = jnp.zeros((1, D), jnp.float32)

    h = x
    hp = _pack_x(x)
    for i in range(3):
        ws = Wl[i][:, :D].T
        wn = Wl[i][:, D:].T * (1.0 / K)
        bias = bl[i].reshape(1, D)
        table = hp.reshape(N, DW)
        aggsum = _neighbor_sum(table, idx).reshape(NPAD, D)
        if i < 2:
            h, hp = _tc_layer(h, aggsum, ws, wn, bias, ln_g[i].reshape(1, D),
                              ln_b[i].reshape(1, D), True, True)
        else:
            h, _ = _tc_layer(h, aggsum, ws, wn, bias, one, zero, False,
                             False)
    return h
